# baseline (speedup 1.0000x reference)
import jax
import jax.numpy as jnp
from jax import lax
from jax.experimental import pallas as pl
from jax.experimental.pallas import tpu as pltpu
from jax.experimental.pallas import tpu_sc as plsc

N_NODES = 10000
N_EDGES = 160000
HIDDEN = 256
NUM_REL = 1024
NPACK = HIDDEN // 32
HID_W = HIDDEN // 2

NW = 32
E_PER_W = N_EDGES // NW
CHUNK = 96
N_FULL = E_PER_W // CHUNK
TAIL = E_PER_W - N_FULL * CHUNK
TAIL_G = (TAIL + 15) // 16
OUT_PAD = N_FULL * CHUNK + TAIL_G * 16

Z_RPW = 312
Z_RCHUNK = 104
Z_LEFT = N_NODES - NW * Z_RPW
REL_RPW = NUM_REL // NW
EI_BLK = 4992
EI_LAST = N_EDGES - 31 * EI_BLK


def _pack_kernel(z_hbm, rel_hbm, ei_hbm, zp_hbm, rp_hbm, srco_hbm, dsto_hbm,
                 zin, zout, rin, rout, eib):
    wid = lax.axis_index("s") * 2 + lax.axis_index("c")

    def pack_rows(src_v, dst_v, n_rows):
        @plsc.parallel_loop(0, n_rows, unroll=2)
        def _row(r):
            for g in range(NPACK):
                a = src_v[r, pl.ds(g * 32, 16)]
                b = src_v[r, pl.ds(g * 32 + 16, 16)]
                w = plsc.bitcast(
                    plsc.pack(a, b, format=plsc.PackFormat.INTERLEAVED),
                    jnp.int32)
                dst_v[r, pl.ds(g * 16, 16)] = w

    def z_chunk(c, _):
        r0 = wid * Z_RPW + c * Z_RCHUNK
        pltpu.sync_copy(z_hbm.at[pl.ds(r0, Z_RCHUNK)], zin)
        pack_rows(zin, zout, Z_RCHUNK)
        pltpu.sync_copy(zout, zp_hbm.at[pl.ds(r0, Z_RCHUNK)])
        return 0

    lax.fori_loop(0, Z_RPW // Z_RCHUNK, z_chunk, 0)

    @pl.when(wid < 2)
    def _leftover():
        r0 = NW * Z_RPW + wid * (Z_LEFT // 2)
        pltpu.sync_copy(z_hbm.at[pl.ds(r0, Z_LEFT // 2)],
                        zin.at[pl.ds(0, Z_LEFT // 2)])
        pack_rows(zin, zout, Z_LEFT // 2)
        pltpu.sync_copy(zout.at[pl.ds(0, Z_LEFT // 2)],
                        zp_hbm.at[pl.ds(r0, Z_LEFT // 2)])

    rr0 = wid * REL_RPW
    pltpu.sync_copy(rel_hbm.at[pl.ds(rr0, REL_RPW)], rin)
    pack_rows(rin, rout, REL_RPW)
    pltpu.sync_copy(rout, rp_hbm.at[pl.ds(rr0, REL_RPW)])

    def split_ei(off, n):
        pltpu.sync_copy(ei_hbm.at[:, pl.ds(off, n)], eib.at[:, pl.ds(0, n)])
        pltpu.sync_copy(eib.at[0, pl.ds(0, n)], srco_hbm.at[pl.ds(off, n)])
        pltpu.sync_copy(eib.at[1, pl.ds(0, n)], dsto_hbm.at[pl.ds(off, n)])

    @pl.when(wid < NW - 1)
    def _split_main():
        split_ei(wid * EI_BLK, EI_BLK)

    @pl.when(wid == NW - 1)
    def _split_last():
        split_ei((NW - 1) * EI_BLK, EI_LAST)


@jax.jit
def _pack_tables(z, rel_emb, ei):
    mesh = plsc.VectorSubcoreMesh(core_axis_name="c", subcore_axis_name="s")
    f = pl.kernel(
        _pack_kernel,
        out_type=[
            jax.ShapeDtypeStruct((N_NODES, HID_W), jnp.int32),
            jax.ShapeDtypeStruct((NUM_REL, HID_W), jnp.int32),
            jax.ShapeDtypeStruct((N_EDGES,), jnp.int32),
            jax.ShapeDtypeStruct((N_EDGES,), jnp.int32),
        ],
        mesh=mesh,
        scratch_types=[
            pltpu.VMEM((Z_RCHUNK, HIDDEN), jnp.float32),
            pltpu.VMEM((Z_RCHUNK, HID_W), jnp.int32),
            pltpu.VMEM((REL_RPW, HIDDEN), jnp.float32),
            pltpu.VMEM((REL_RPW, HID_W), jnp.int32),
            pltpu.VMEM((2, EI_LAST), jnp.int32),
        ],
        compiler_params=pltpu.CompilerParams(needs_layout_passes=False),
    )
    return f(z, rel_emb, ei)


def _sc_kernel(src_hbm, dst_hbm, typ_hbm, z_hbm, rel_hbm, out_hbm,
               idx_all_s, idx_all_d, idx_all_t,
               rs0, rd0, rr0, rs1, rd1, rr1,
               part, out_v, sem0, sem1):
    wid = lax.axis_index("s") * 2 + lax.axis_index("c")
    base = wid * E_PER_W
    lanes16 = lax.iota(jnp.int32, 16) * 16

    def issue(c, rs, rd, rr, sem):
        o = c * CHUNK
        pltpu.async_copy(z_hbm.at[idx_all_s.at[pl.ds(o, CHUNK)]], rs, sem)
        pltpu.async_copy(z_hbm.at[idx_all_d.at[pl.ds(o, CHUNK)]], rd, sem)
        pltpu.async_copy(rel_hbm.at[idx_all_t.at[pl.ds(o, CHUNK)]], rr, sem)

    def drain(rs, rd, rr, sem):
        pltpu.make_async_copy(z_hbm.at[idx_all_s.at[pl.ds(0, CHUNK)]], rs, sem).wait()
        pltpu.make_async_copy(z_hbm.at[idx_all_d.at[pl.ds(0, CHUNK)]], rd, sem).wait()
        pltpu.make_async_copy(rel_hbm.at[idx_all_t.at[pl.ds(0, CHUNK)]], rr, sem).wait()

    def compute(rs, rd, rr, out_off, n_edges, n_groups):
        def prod(e, k):
            sl = pl.ds(k * 16, 16)
            s = plsc.bitcast(rs[e, sl], jnp.bfloat16)
            r = plsc.bitcast(rr[e, sl], jnp.bfloat16)
            d = plsc.bitcast(rd[e, sl], jnp.bfloat16)
            return s * r * d

        def unpk(p):
            return plsc.unpack(p, format=plsc.PackFormat.INTERLEAVED,
                               preferred_element_type=jnp.float32)

        @plsc.parallel_loop(0, n_edges, unroll=4)
        def _edge_body(e):
            a0, a1 = unpk(prod(e, 0))
            a2, a3 = unpk(prod(e, 1))
            for k in range(2, NPACK):
                x, y = unpk(prod(e, k))
                if k % 2 == 0:
                    a0, a1 = a0 + x, a1 + y
                else:
                    a2, a3 = a2 + x, a3 + y
            part[pl.ds(e * 16, 16)] = (a0 + a2) + (a1 + a3)

        for g in range(n_groups):
            acc = plsc.load_gather(part, [lanes16 + g * 256])
            for k in range(1, 16):
                acc = acc + plsc.load_gather(part, [lanes16 + (g * 256 + k)])
            out_v[pl.ds(out_off + g * 16, 16)] = acc

    pltpu.sync_copy(src_hbm.at[pl.ds(base, E_PER_W)], idx_all_s)
    pltpu.sync_copy(dst_hbm.at[pl.ds(base, E_PER_W)], idx_all_d)
    pltpu.sync_copy(typ_hbm.at[pl.ds(base, E_PER_W)], idx_all_t)

    issue(0, rs0, rd0, rr0, sem0)
    issue(1, rs1, rd1, rr1, sem1)

    def pair_body(i, _):
        c = i * 2
        drain(rs0, rd0, rr0, sem0)
        compute(rs0, rd0, rr0, c * CHUNK, CHUNK, CHUNK // 16)
        issue(c + 2, rs0, rd0, rr0, sem0)
        drain(rs1, rd1, rr1, sem1)
        compute(rs1, rd1, rr1, (c + 1) * CHUNK, CHUNK, CHUNK // 16)
        issue(c + 3, rs1, rd1, rr1, sem1)
        return 0

    lax.fori_loop(0, (N_FULL - 2) // 2, pair_body, 0)

    drain(rs0, rd0, rr0, sem0)
    compute(rs0, rd0, rr0, (N_FULL - 2) * CHUNK, CHUNK, CHUNK // 16)
    toff = N_FULL * CHUNK
    pltpu.async_copy(z_hbm.at[idx_all_s.at[pl.ds(toff, TAIL)]],
                     rs0.at[pl.ds(0, TAIL)], sem0)
    pltpu.async_copy(z_hbm.at[idx_all_d.at[pl.ds(toff, TAIL)]],
                     rd0.at[pl.ds(0, TAIL)], sem0)
    pltpu.async_copy(rel_hbm.at[idx_all_t.at[pl.ds(toff, TAIL)]],
                     rr0.at[pl.ds(0, TAIL)], sem0)

    drain(rs1, rd1, rr1, sem1)
    compute(rs1, rd1, rr1, (N_FULL - 1) * CHUNK, CHUNK, CHUNK // 16)

    pltpu.make_async_copy(z_hbm.at[idx_all_s.at[pl.ds(toff, TAIL)]],
                          rs0.at[pl.ds(0, TAIL)], sem0).wait()
    pltpu.make_async_copy(z_hbm.at[idx_all_d.at[pl.ds(toff, TAIL)]],
                          rd0.at[pl.ds(0, TAIL)], sem0).wait()
    pltpu.make_async_copy(rel_hbm.at[idx_all_t.at[pl.ds(toff, TAIL)]],
                          rr0.at[pl.ds(0, TAIL)], sem0).wait()
    compute(rs0, rd0, rr0, toff, TAIL, TAIL_G)

    pltpu.sync_copy(out_v.at[pl.ds(0, E_PER_W)],
                    out_hbm.at[pl.ds(base, E_PER_W)])


@jax.jit
def _dist_mult(src, dst, typ, z, rel_emb):
    mesh = plsc.VectorSubcoreMesh(core_axis_name="c", subcore_axis_name="s")
    rows = pltpu.VMEM((CHUNK, HID_W), jnp.int32)
    f = pl.kernel(
        _sc_kernel,
        out_type=jax.ShapeDtypeStruct((N_EDGES,), jnp.float32),
        mesh=mesh,
        scratch_types=[
            pltpu.VMEM((E_PER_W,), jnp.int32),
            pltpu.VMEM((E_PER_W,), jnp.int32),
            pltpu.VMEM((E_PER_W,), jnp.int32),
            rows, rows, rows, rows, rows, rows,
            pltpu.VMEM((CHUNK * 16,), jnp.float32),
            pltpu.VMEM((OUT_PAD,), jnp.float32),
            pltpu.SemaphoreType.DMA,
            pltpu.SemaphoreType.DMA,
        ],
        compiler_params=pltpu.CompilerParams(needs_layout_passes=False),
    )
    return f(src, dst, typ, z, rel_emb)


def kernel(z, edge_index, edge_type, rel_emb):
    edge_index = edge_index.astype(jnp.int32)
    edge_type = edge_type.astype(jnp.int32)
    z_p, rel_p, src, dst = _pack_tables(z, rel_emb, edge_index)
    return _dist_mult(src, dst, edge_type, z_p, rel_p)


# speedup vs baseline: 1.0564x; 1.0564x over previous
import jax
import jax.numpy as jnp
from jax import lax
from jax.experimental import pallas as pl
from jax.experimental.pallas import tpu as pltpu
from jax.experimental.pallas import tpu_sc as plsc

N_NODES = 10000
N_EDGES = 160000
HIDDEN = 256
NUM_REL = 1024
NPACK = HIDDEN // 32
HID_W = HIDDEN // 2

NW = 32
E_PER_W = N_EDGES // NW
CHUNK = 80
N_FULL = E_PER_W // CHUNK
TAIL = E_PER_W - N_FULL * CHUNK
TAIL_G = (TAIL + 15) // 16
OUT_PAD = N_FULL * CHUNK + TAIL_G * 16

Z_RPW = 312
Z_RCHUNK = 104
Z_LEFT = N_NODES - NW * Z_RPW
REL_RPW = NUM_REL // NW


def _pack_kernel(z_hbm, rel_hbm, zp_hbm, rp_hbm, zin, zout, rin, rout):
    wid = lax.axis_index("s") * 2 + lax.axis_index("c")

    def pack_rows(src_v, dst_v, n_rows):
        @plsc.parallel_loop(0, n_rows, unroll=2)
        def _row(r):
            for g in range(NPACK):
                a = src_v[r, pl.ds(g * 32, 16)]
                b = src_v[r, pl.ds(g * 32 + 16, 16)]
                w = plsc.bitcast(
                    plsc.pack(a, b, format=plsc.PackFormat.INTERLEAVED),
                    jnp.int32)
                dst_v[r, pl.ds(g * 16, 16)] = w

    def z_chunk(c, _):
        r0 = wid * Z_RPW + c * Z_RCHUNK
        pltpu.sync_copy(z_hbm.at[pl.ds(r0, Z_RCHUNK)], zin)
        pack_rows(zin, zout, Z_RCHUNK)
        pltpu.sync_copy(zout, zp_hbm.at[pl.ds(r0, Z_RCHUNK)])
        return 0

    lax.fori_loop(0, Z_RPW // Z_RCHUNK, z_chunk, 0)

    @pl.when(wid < 2)
    def _leftover():
        r0 = NW * Z_RPW + wid * (Z_LEFT // 2)
        pltpu.sync_copy(z_hbm.at[pl.ds(r0, Z_LEFT // 2)],
                        zin.at[pl.ds(0, Z_LEFT // 2)])
        pack_rows(zin, zout, Z_LEFT // 2)
        pltpu.sync_copy(zout.at[pl.ds(0, Z_LEFT // 2)],
                        zp_hbm.at[pl.ds(r0, Z_LEFT // 2)])

    rr0 = wid * REL_RPW
    pltpu.sync_copy(rel_hbm.at[pl.ds(rr0, REL_RPW)], rin)
    pack_rows(rin, rout, REL_RPW)
    pltpu.sync_copy(rout, rp_hbm.at[pl.ds(rr0, REL_RPW)])


@jax.jit
def _pack_tables(z, rel_emb):
    mesh = plsc.VectorSubcoreMesh(core_axis_name="c", subcore_axis_name="s")
    f = pl.kernel(
        _pack_kernel,
        out_type=[
            jax.ShapeDtypeStruct((N_NODES, HID_W), jnp.int32),
            jax.ShapeDtypeStruct((NUM_REL, HID_W), jnp.int32),
        ],
        mesh=mesh,
        scratch_types=[
            pltpu.VMEM((Z_RCHUNK, HIDDEN), jnp.float32),
            pltpu.VMEM((Z_RCHUNK, HID_W), jnp.int32),
            pltpu.VMEM((REL_RPW, HIDDEN), jnp.float32),
            pltpu.VMEM((REL_RPW, HID_W), jnp.int32),
        ],
        compiler_params=pltpu.CompilerParams(needs_layout_passes=False),
    )
    return f(z, rel_emb)


def _sc_kernel(src_hbm, dst_hbm, typ_hbm, z_hbm, rel_hbm, out_hbm,
               idx_all_s, idx_all_d, idx_all_t,
               rs0, rd0, rr0, rs1, rd1, rr1, rs2, rd2, rr2,
               part, out_v, sem0, sem1, sem2):
    wid = lax.axis_index("s") * 2 + lax.axis_index("c")
    base = wid * E_PER_W
    lanes16 = lax.iota(jnp.int32, 16) * 16

    def issue(c, rs, rd, rr, sem):
        o = c * CHUNK
        pltpu.async_copy(z_hbm.at[idx_all_s.at[pl.ds(o, CHUNK)]], rs, sem)
        pltpu.async_copy(z_hbm.at[idx_all_d.at[pl.ds(o, CHUNK)]], rd, sem)
        pltpu.async_copy(rel_hbm.at[idx_all_t.at[pl.ds(o, CHUNK)]], rr, sem)

    def drain(rs, rd, rr, sem):
        pltpu.make_async_copy(z_hbm.at[idx_all_s.at[pl.ds(0, CHUNK)]], rs, sem).wait()
        pltpu.make_async_copy(z_hbm.at[idx_all_d.at[pl.ds(0, CHUNK)]], rd, sem).wait()
        pltpu.make_async_copy(rel_hbm.at[idx_all_t.at[pl.ds(0, CHUNK)]], rr, sem).wait()

    def compute(rs, rd, rr, out_off, n_edges, n_groups):
        def prod(e, k):
            sl = pl.ds(k * 16, 16)
            s = plsc.bitcast(rs[e, sl], jnp.bfloat16)
            r = plsc.bitcast(rr[e, sl], jnp.bfloat16)
            d = plsc.bitcast(rd[e, sl], jnp.bfloat16)
            return s * r * d

        def unpk(p):
            return plsc.unpack(p, format=plsc.PackFormat.INTERLEAVED,
                               preferred_element_type=jnp.float32)

        @plsc.parallel_loop(0, n_edges, unroll=4)
        def _edge_body(e):
            a0, a1 = unpk(prod(e, 0))
            a2, a3 = unpk(prod(e, 1))
            for k in range(2, NPACK):
                x, y = unpk(prod(e, k))
                if k % 2 == 0:
                    a0, a1 = a0 + x, a1 + y
                else:
                    a2, a3 = a2 + x, a3 + y
            part[pl.ds(e * 16, 16)] = (a0 + a2) + (a1 + a3)

        for g in range(n_groups):
            acc = plsc.load_gather(part, [lanes16 + g * 256])
            for k in range(1, 16):
                acc = acc + plsc.load_gather(part, [lanes16 + (g * 256 + k)])
            out_v[pl.ds(out_off + g * 16, 16)] = acc

    pltpu.sync_copy(src_hbm.at[pl.ds(base, E_PER_W)], idx_all_s)
    pltpu.sync_copy(dst_hbm.at[pl.ds(base, E_PER_W)], idx_all_d)
    pltpu.sync_copy(typ_hbm.at[pl.ds(base, E_PER_W)], idx_all_t)

    issue(0, rs0, rd0, rr0, sem0)
    issue(1, rs1, rd1, rr1, sem1)
    issue(2, rs2, rd2, rr2, sem2)

    def tri_body(i, _):
        c = i * 3
        drain(rs0, rd0, rr0, sem0)
        compute(rs0, rd0, rr0, c * CHUNK, CHUNK, CHUNK // 16)
        issue(c + 3, rs0, rd0, rr0, sem0)
        drain(rs1, rd1, rr1, sem1)
        compute(rs1, rd1, rr1, (c + 1) * CHUNK, CHUNK, CHUNK // 16)
        issue(c + 4, rs1, rd1, rr1, sem1)
        drain(rs2, rd2, rr2, sem2)
        compute(rs2, rd2, rr2, (c + 2) * CHUNK, CHUNK, CHUNK // 16)
        issue(c + 5, rs2, rd2, rr2, sem2)
        return 0

    lax.fori_loop(0, (N_FULL - 5) // 3, tri_body, 0)

    toff = N_FULL * CHUNK
    c = N_FULL - 5
    drain(rs0, rd0, rr0, sem0)
    compute(rs0, rd0, rr0, c * CHUNK, CHUNK, CHUNK // 16)
    issue(c + 3, rs0, rd0, rr0, sem0)
    drain(rs1, rd1, rr1, sem1)
    compute(rs1, rd1, rr1, (c + 1) * CHUNK, CHUNK, CHUNK // 16)
    issue(c + 4, rs1, rd1, rr1, sem1)
    drain(rs2, rd2, rr2, sem2)
    compute(rs2, rd2, rr2, (c + 2) * CHUNK, CHUNK, CHUNK // 16)
    pltpu.async_copy(z_hbm.at[idx_all_s.at[pl.ds(toff, TAIL)]],
                     rs2.at[pl.ds(0, TAIL)], sem2)
    pltpu.async_copy(z_hbm.at[idx_all_d.at[pl.ds(toff, TAIL)]],
                     rd2.at[pl.ds(0, TAIL)], sem2)
    pltpu.async_copy(rel_hbm.at[idx_all_t.at[pl.ds(toff, TAIL)]],
                     rr2.at[pl.ds(0, TAIL)], sem2)

    drain(rs0, rd0, rr0, sem0)
    compute(rs0, rd0, rr0, (c + 3) * CHUNK, CHUNK, CHUNK // 16)
    drain(rs1, rd1, rr1, sem1)
    compute(rs1, rd1, rr1, (c + 4) * CHUNK, CHUNK, CHUNK // 16)

    pltpu.make_async_copy(z_hbm.at[idx_all_s.at[pl.ds(toff, TAIL)]],
                          rs2.at[pl.ds(0, TAIL)], sem2).wait()
    pltpu.make_async_copy(z_hbm.at[idx_all_d.at[pl.ds(toff, TAIL)]],
                          rd2.at[pl.ds(0, TAIL)], sem2).wait()
    pltpu.make_async_copy(rel_hbm.at[idx_all_t.at[pl.ds(toff, TAIL)]],
                          rr2.at[pl.ds(0, TAIL)], sem2).wait()
    compute(rs2, rd2, rr2, toff, TAIL, TAIL_G)

    pltpu.sync_copy(out_v.at[pl.ds(0, E_PER_W)],
                    out_hbm.at[pl.ds(base, E_PER_W)])


@jax.jit
def _dist_mult(src, dst, typ, z, rel_emb):
    mesh = plsc.VectorSubcoreMesh(core_axis_name="c", subcore_axis_name="s")
    rows = pltpu.VMEM((CHUNK, HID_W), jnp.int32)
    f = pl.kernel(
        _sc_kernel,
        out_type=jax.ShapeDtypeStruct((N_EDGES,), jnp.float32),
        mesh=mesh,
        scratch_types=[
            pltpu.VMEM((E_PER_W,), jnp.int32),
            pltpu.VMEM((E_PER_W,), jnp.int32),
            pltpu.VMEM((E_PER_W,), jnp.int32),
            rows, rows, rows, rows, rows, rows, rows, rows, rows,
            pltpu.VMEM((CHUNK * 16,), jnp.float32),
            pltpu.VMEM((OUT_PAD,), jnp.float32),
            pltpu.SemaphoreType.DMA,
            pltpu.SemaphoreType.DMA,
            pltpu.SemaphoreType.DMA,
        ],
        compiler_params=pltpu.CompilerParams(needs_layout_passes=False),
    )
    return f(src, dst, typ, z, rel_emb)


def kernel(z, edge_index, edge_type, rel_emb):
    edge_index = edge_index.astype(jnp.int32)
    edge_type = edge_type.astype(jnp.int32)
    z_p, rel_p = _pack_tables(z, rel_emb)
    return _dist_mult(edge_index[0], edge_index[1], edge_type, z_p, rel_p)
